# baseline (speedup 1.0000x reference)
import functools
import jax
import jax.numpy as jnp
from jax import lax
from jax.experimental import pallas as pl
from jax.experimental.pallas import tpu as pltpu

_N = 320000
_D_IN = 128
_D_HID = 64
_NUM_SEG = 10000
_EPS = 1e-3
_ROWS_BLK = 3200


def _mm_stats_body(x_ref, w_ref, g_ref, b_ref, y_ref, par_ref, acc_ref):
    i = pl.program_id(0)
    xb = x_ref[...]
    w = w_ref[...]
    y = lax.dot_general(xb, w, (((1,), (1,)), ((), ())),
                        preferred_element_type=jnp.float32)
    y_ref[...] = y

    @pl.when(i == 0)
    def _():
        acc_ref[...] = jnp.zeros_like(acc_ref)

    acc_ref[0:1, :] += jnp.sum(y, axis=0, keepdims=True)
    acc_ref[1:2, :] += jnp.sum(y * y, axis=0, keepdims=True)

    @pl.when(i == pl.num_programs(0) - 1)
    def _():
        mean = acc_ref[0:1, :] / _N
        var = acc_ref[1:2, :] / _N - mean * mean
        scale = g_ref[...] * lax.rsqrt(var + _EPS)
        shift = b_ref[...] - mean * scale
        par_ref[0:1, :] = scale
        par_ref[1:2, :] = shift


def _mm_stats(inputs, W, gamma, beta):
    nblk = _N // _ROWS_BLK
    y, par = pl.pallas_call(
        _mm_stats_body,
        grid=(nblk,),
        in_specs=[
            pl.BlockSpec((_ROWS_BLK, _D_IN), lambda i: (i, 0)),
            pl.BlockSpec((_D_HID, _D_IN), lambda i: (0, 0)),
            pl.BlockSpec((1, _D_HID), lambda i: (0, 0)),
            pl.BlockSpec((1, _D_HID), lambda i: (0, 0)),
        ],
        out_specs=[
            pl.BlockSpec((_ROWS_BLK, _D_HID), lambda i: (i, 0)),
            pl.BlockSpec((2, _D_HID), lambda i: (0, 0)),
        ],
        out_shape=[
            jax.ShapeDtypeStruct((_N, _D_HID), jnp.float32),
            jax.ShapeDtypeStruct((2, _D_HID), jnp.float32),
        ],
        scratch_shapes=[pltpu.VMEM((2, _D_HID), jnp.float32)],
    )(inputs, W, gamma.reshape(1, -1), beta.reshape(1, -1))
    return y, par


def kernel(inputs, unq_inv, W, gamma, beta):
    x_raw, par = _mm_stats(inputs, W, gamma, beta)
    scale = par[0]
    shift = par[1]
    x = x_raw * scale + shift
    x_max = jax.ops.segment_max(x, unq_inv, num_segments=_NUM_SEG)
    x_min = jax.ops.segment_min(x, unq_inv, num_segments=_NUM_SEG)
    x_out = jnp.concatenate([x_min, x_max], axis=1)
    return jnp.concatenate([x, x_out[unq_inv, :]], axis=1)


# speedup vs baseline: 2.4250x; 2.4250x over previous
import dataclasses
import functools
import jax
import jax.numpy as jnp
from jax import lax
from jax.experimental import pallas as pl
from jax.experimental.pallas import tpu as pltpu
from jax.experimental.pallas import tpu_sc as plsc

_N = 320000
_D_IN = 128
_D_HID = 64
_NUM_SEG = 10000
_EPS = 1e-3

_ROWS_BLK = 3200
_NT = 32
_TROWS = _N // _NT
_CHUNK = 400
_NCHUNK = _TROWS // _CHUNK
_STAGE = 64
_D2 = 2 * _D_HID

_NEG = -3.4028235e38
_POS = 3.4028235e38


def _sc_params():
    cp = pltpu.CompilerParams()
    if "needs_layout_passes" in pltpu.CompilerParams.__dataclass_fields__:
        cp = dataclasses.replace(cp, needs_layout_passes=False)
    return cp


def _mm_body(x_ref, w_ref, g_ref, b_ref, y_ref, par_ref, acc_ref):
    i = pl.program_id(0)
    y = lax.dot_general(x_ref[...], w_ref[...], (((1,), (1,)), ((), ())),
                        preferred_element_type=jnp.float32)
    y_ref[...] = y

    @pl.when(i == 0)
    def _():
        acc_ref[...] = jnp.zeros_like(acc_ref)

    acc_ref[0:1, :] += jnp.sum(y, axis=0, keepdims=True)
    acc_ref[1:2, :] += jnp.sum(y * y, axis=0, keepdims=True)

    @pl.when(i == pl.num_programs(0) - 1)
    def _():
        mean = acc_ref[0:1, :] / _N
        var = acc_ref[1:2, :] / _N - mean * mean
        scale = g_ref[...] * lax.rsqrt(var + _EPS)
        par_ref[0:1, :] = scale
        par_ref[1:2, :] = b_ref[...] - mean * scale


def _mm_stats(inputs, W, gamma, beta):
    nblk = _N // _ROWS_BLK
    return pl.pallas_call(
        _mm_body,
        grid=(nblk,),
        in_specs=[
            pl.BlockSpec((_ROWS_BLK, _D_IN), lambda i: (i, 0)),
            pl.BlockSpec((_D_HID, _D_IN), lambda i: (0, 0)),
            pl.BlockSpec((1, _D_HID), lambda i: (0, 0)),
            pl.BlockSpec((1, _D_HID), lambda i: (0, 0)),
        ],
        out_specs=[
            pl.BlockSpec((_ROWS_BLK, _D_HID), lambda i: (i, 0)),
            pl.BlockSpec((2, _D_HID), lambda i: (0, 0)),
        ],
        out_shape=[
            jax.ShapeDtypeStruct((_N, _D_HID), jnp.float32),
            jax.ShapeDtypeStruct((2, _D_HID), jnp.float32),
        ],
        scratch_shapes=[pltpu.VMEM((2, _D_HID), jnp.float32)],
    )(inputs, W, gamma.reshape(1, -1), beta.reshape(1, -1))


def _seg_reduce_body(xbn, ids, xout, recvals, recids,
                     buf, idsb, stage, rec_v, accv, recid_v, st_sm,
                     sem0, sem1, sem_out, sem_fin):
    t = lax.axis_index("s") * 2 + lax.axis_index("c")
    r0 = t * _TROWS
    _GPC = _CHUNK // 16

    def issue(ci):
        half = lax.rem(ci, 2)
        src_x = xbn.at[pl.ds((r0 + ci * _CHUNK) * _D_HID, _CHUNK * _D_HID)]
        dst_x = buf.at[pl.ds(half * _CHUNK * _D_HID, _CHUNK * _D_HID)]
        src_i = ids.at[pl.ds(r0 + ci * _CHUNK, _CHUNK)]
        dst_i = idsb.at[pl.ds(half * _CHUNK, _CHUNK)]

        @pl.when(half == 0)
        def _():
            pltpu.async_copy(src_x, dst_x, sem0)
            pltpu.async_copy(src_i, dst_i, sem0)

        @pl.when(half == 1)
        def _():
            pltpu.async_copy(src_x, dst_x, sem1)
            pltpu.async_copy(src_i, dst_i, sem1)

    def wait(ci):
        dummy_x = buf.at[pl.ds(0, _CHUNK * _D_HID)]
        dummy_i = idsb.at[pl.ds(0, _CHUNK)]
        src_x = xbn.at[pl.ds(0, _CHUNK * _D_HID)]
        src_i = ids.at[pl.ds(0, _CHUNK)]

        @pl.when(lax.rem(ci, 2) == 0)
        def _():
            pltpu.make_async_copy(src_x, dummy_x, sem0).wait()
            pltpu.make_async_copy(src_i, dummy_i, sem0).wait()

        @pl.when(lax.rem(ci, 2) == 1)
        def _():
            pltpu.make_async_copy(src_x, dummy_x, sem1).wait()
            pltpu.make_async_copy(src_i, dummy_i, sem1).wait()

    issue(jnp.int32(0))
    issue(jnp.int32(1))

    st_sm[0] = jnp.int32(-1)
    st_sm[1] = jnp.int32(0)
    st_sm[2] = jnp.int32(-1)
    for k in range(4):
        accv[pl.ds(16 * k, 16)] = jnp.full((16,), _POS, jnp.float32)
        accv[pl.ds(_D_HID + 16 * k, 16)] = jnp.full((16,), _NEG, jnp.float32)

    def flush_seg(cur, nf, amin, amax):
        m = nf - 1

        @pl.when(nf == 0)
        def _():
            for k in range(4):
                rec_v[pl.ds(16 * k, 16)] = amin[k]
                rec_v[pl.ds(_D_HID + 16 * k, 16)] = amax[k]

        @pl.when(nf > 0)
        def _():
            @pl.when(jnp.logical_and(m > 0, lax.rem(m, _STAGE) == 0))
            def _():
                def dr(i, z):
                    pltpu.make_async_copy(
                        xout.at[0], stage.at[pl.ds(0, _D2)], sem_out).wait()
                    return z
                lax.fori_loop(0, _STAGE, dr, 0)
            jb = lax.rem(jnp.maximum(m, 0), _STAGE) * _D2
            for k in range(4):
                stage[pl.ds(jb + 16 * k, 16)] = amin[k]
                stage[pl.ds(jb + _D_HID + 16 * k, 16)] = amax[k]
            pltpu.async_copy(stage.at[pl.ds(jb, _D2)], xout.at[cur], sem_out)

    def chunk(ci, z):
        wait(ci)
        half = lax.rem(ci, 2)
        ibase = half * _CHUNK
        xbase = half * _CHUNK * _D_HID

        def group(g, zz):
            gib = ibase + g * 16
            gxb = xbase + g * 16 * _D_HID
            idvec = idsb[pl.ds(gib, 16)]
            i0 = idvec[0]
            i15 = idvec[15]
            cur0 = st_sm[0]
            fast = jnp.logical_and(i0 == cur0, i15 == i0)

            @pl.when(fast)
            def _():
                amin = [accv[pl.ds(16 * k, 16)] for k in range(4)]
                amax = [accv[pl.ds(_D_HID + 16 * k, 16)] for k in range(4)]
                for l in range(16):
                    base = gxb + l * _D_HID
                    for k in range(4):
                        xk = buf[pl.ds(base + 16 * k, 16)]
                        amin[k] = jnp.minimum(amin[k], xk)
                        amax[k] = jnp.maximum(amax[k], xk)
                for k in range(4):
                    accv[pl.ds(16 * k, 16)] = amin[k]
                    accv[pl.ds(_D_HID + 16 * k, 16)] = amax[k]

            @pl.when(jnp.logical_not(fast))
            def _():
                cur, nf, fid = st_sm[0], st_sm[1], st_sm[2]
                amin = [accv[pl.ds(16 * k, 16)] for k in range(4)]
                amax = [accv[pl.ds(_D_HID + 16 * k, 16)] for k in range(4)]
                first_g = jnp.logical_and(ci == 0, g == 0)
                for l in range(16):
                    idv = idvec[l]
                    base = gxb + l * _D_HID
                    xs = [buf[pl.ds(base + 16 * k, 16)] for k in range(4)]
                    brk = idv != cur
                    if l == 0:
                        brk = jnp.logical_and(brk, jnp.logical_not(first_g))

                    @pl.when(brk)
                    def _(cur=cur, nf=nf, amin=amin, amax=amax):
                        flush_seg(cur, nf, amin, amax)

                    amin = [jnp.where(brk, xs[k], jnp.minimum(amin[k], xs[k]))
                            for k in range(4)]
                    amax = [jnp.where(brk, xs[k], jnp.maximum(amax[k], xs[k]))
                            for k in range(4)]
                    fid = jnp.where(jnp.logical_and(brk, nf == 0), cur, fid)
                    nf = nf + brk.astype(jnp.int32)
                    cur = idv
                for k in range(4):
                    accv[pl.ds(16 * k, 16)] = amin[k]
                    accv[pl.ds(_D_HID + 16 * k, 16)] = amax[k]
                st_sm[0] = cur
                st_sm[1] = nf
                st_sm[2] = fid
            return zz

        z = lax.fori_loop(0, _GPC, group, z)

        @pl.when(ci + 2 < _NCHUNK)
        def _():
            issue(ci + 2)
        return z

    lax.fori_loop(0, _NCHUNK, chunk, 0)

    cur, nf, fid = st_sm[0], st_sm[1], st_sm[2]
    issued = jnp.maximum(nf - 1, 0)

    for k in range(4):
        rec_v[pl.ds(_D2 + 16 * k, 16)] = accv[pl.ds(16 * k, 16)]
        rec_v[pl.ds(_D2 + _D_HID + 16 * k, 16)] = \
            accv[pl.ds(_D_HID + 16 * k, 16)]
    iota = lax.iota(jnp.int32, 16)
    recid_v[...] = jnp.where(iota == 0, fid, jnp.where(iota == 1, cur, 0))

    ndrain = issued - lax.div(jnp.maximum(issued - 1, 0),
                              jnp.int32(_STAGE)) * _STAGE
    ndrain = jnp.where(issued == 0, 0, ndrain)

    def drain(i, z):
        pltpu.make_async_copy(xout.at[0], stage.at[pl.ds(0, _D2)],
                              sem_out).wait()
        return z

    lax.fori_loop(0, ndrain, drain, 0)

    pltpu.async_copy(rec_v, recvals.at[pl.ds(2 * t * _D2, 2 * _D2)], sem_fin)
    pltpu.async_copy(recid_v, recids.at[pl.ds(t * 16, 16)], sem_fin)
    pltpu.make_async_copy(rec_v, recvals.at[pl.ds(2 * t * _D2, 2 * _D2)],
                          sem_fin).wait()
    pltpu.make_async_copy(recid_v, recids.at[pl.ds(t * 16, 16)],
                          sem_fin).wait()


def _seg_reduce(x_bn_flat, unq_inv):
    mesh = plsc.VectorSubcoreMesh(core_axis_name="c", subcore_axis_name="s")
    f = pl.kernel(
        _seg_reduce_body,
        out_type=[
            jax.ShapeDtypeStruct((_NUM_SEG, _D2), jnp.float32),
            jax.ShapeDtypeStruct((2 * _NT * _D2,), jnp.float32),
            jax.ShapeDtypeStruct((_NT * 16,), jnp.int32),
        ],
        mesh=mesh,
        scratch_types=[
            pltpu.VMEM((2 * _CHUNK * _D_HID,), jnp.float32),
            pltpu.VMEM((2 * _CHUNK,), jnp.int32),
            pltpu.VMEM((_STAGE * _D2,), jnp.float32),
            pltpu.VMEM((2 * _D2,), jnp.float32),
            pltpu.VMEM((_D2,), jnp.float32),
            pltpu.VMEM((16,), jnp.int32),
            pltpu.SMEM((8,), jnp.int32),
            pltpu.SemaphoreType.DMA,
            pltpu.SemaphoreType.DMA,
            pltpu.SemaphoreType.DMA,
            pltpu.SemaphoreType.DMA,
        ],
        compiler_params=_sc_params(),
    )
    return f(x_bn_flat, unq_inv)


def _gather_body(ids, xout, recvals, recids, gath,
                 gbuf0, gbuf1, idsv0, idsv1, rv, cfl, recidv,
                 semg0, semg1, semi0, semi1, semo, semr, semf):
    t = lax.axis_index("s") * 2 + lax.axis_index("c")
    r0 = t * _TROWS
    gbufs = (gbuf0, gbuf1)
    idsvs = (idsv0, idsv1)
    semgs = (semg0, semg1)
    semis = (semi0, semi1)

    pltpu.async_copy(recvals, rv, semr)
    pltpu.async_copy(recids, recidv, semr)
    pltpu.make_async_copy(recvals, rv, semr).wait()
    pltpu.make_async_copy(recids, recidv, semr).wait()

    myrec = recidv[pl.ds(t * 16, 16)]
    my_first = myrec[0]
    my_first = jnp.where(my_first == -1, myrec[1], my_first)
    my_last = myrec[1]

    for k in range(4):
        cfl[pl.ds(16 * k, 16)] = jnp.full((16,), _POS, jnp.float32)
        cfl[pl.ds(_D_HID + 16 * k, 16)] = jnp.full((16,), _NEG, jnp.float32)
        cfl[pl.ds(_D2 + 16 * k, 16)] = jnp.full((16,), _POS, jnp.float32)
        cfl[pl.ds(_D2 + _D_HID + 16 * k, 16)] = jnp.full((16,), _NEG,
                                                         jnp.float32)

    def comb(u, z):
        uvec = recidv[pl.ds(u * 16, 16)]
        for pos in range(2):
            uid = uvec[pos]
            ub = (2 * u + pos) * _D2
            for rowb, myid in ((0, my_first), (_D2, my_last)):
                @pl.when(uid == myid)
                def _(ub=ub, rowb=rowb):
                    for k in range(4):
                        cfl[pl.ds(rowb + 16 * k, 16)] = jnp.minimum(
                            cfl[pl.ds(rowb + 16 * k, 16)],
                            rv[pl.ds(ub + 16 * k, 16)])
                        cfl[pl.ds(rowb + _D_HID + 16 * k, 16)] = jnp.maximum(
                            cfl[pl.ds(rowb + _D_HID + 16 * k, 16)],
                            rv[pl.ds(ub + _D_HID + 16 * k, 16)])
        return z

    lax.fori_loop(0, _NT, comb, 0)

    def issue(ci):
        b = ci % 2
        base = r0 + ci * _CHUNK
        pltpu.async_copy(ids.at[pl.ds(base, _CHUNK)], idsvs[b], semis[b])

    def wait_ids(ci):
        b = ci % 2
        pltpu.make_async_copy(ids.at[pl.ds(r0, _CHUNK)], idsvs[b],
                              semis[b]).wait()

    def gather(ci):
        b = ci % 2
        for off, ln in ((0, 128), (128, 128), (256, 128), (384, 16)):
            pltpu.async_copy(xout.at[idsvs[b].at[pl.ds(off, ln)]],
                             gbufs[b].at[pl.ds(off, ln)], semgs[b])

    def wait_gather(ci):
        b = ci % 2
        for off, ln in ((0, 128), (128, 128), (256, 128), (384, 16)):
            pltpu.make_async_copy(
                xout.at[idsvs[b].at[pl.ds(off, ln)]],
                gbufs[b].at[pl.ds(off, ln)], semgs[b]).wait()

    issue(0)
    wait_ids(0)
    gather(0)
    issue(1)

    accf = jnp.zeros((16,), jnp.int32)
    accl = jnp.zeros((16,), jnp.int32)

    for ci in range(_NCHUNK):
        b = ci % 2
        base = r0 + ci * _CHUNK
        gbuf = gbufs[b]
        idsv = idsvs[b]

        def cnt(g, acc, idsv=idsv):
            af, al = acc
            idg = idsv[pl.ds(16 * g, 16)]
            af = af + (idg == my_first).astype(jnp.int32)
            al = al + (idg == my_last).astype(jnp.int32)
            return (af, al)

        accf, accl = lax.fori_loop(0, _CHUNK // 16, cnt, (accf, accl))

        wait_gather(ci)
        if ci + 1 < _NCHUNK:
            wait_ids(ci + 1)
            gather(ci + 1)
        if ci + 2 < _NCHUNK:
            issue(ci + 2)

        pltpu.async_copy(gbuf, gath.at[pl.ds(base, _CHUNK)], semo)
        if ci >= 1:
            pltpu.make_async_copy(
                gbufs[(ci + 1) % 2], gath.at[pl.ds(r0, _CHUNK)], semo).wait()

    pltpu.make_async_copy(
        gbufs[(_NCHUNK - 1) % 2], gath.at[pl.ds(r0, _CHUNK)], semo).wait()

    n_f = jnp.sum(accf)
    n_l = jnp.sum(accl)

    def fixf(i, z):
        pltpu.async_copy(cfl.at[pl.ds(0, _D2)], gath.at[r0 + i], semf)
        return z

    lax.fori_loop(0, n_f, fixf, 0)

    def fixl(i, z):
        pltpu.async_copy(cfl.at[pl.ds(_D2, _D2)],
                         gath.at[r0 + _TROWS - 1 - i], semf)
        return z

    lax.fori_loop(0, n_l, fixl, 0)

    def drainf(i, z):
        pltpu.make_async_copy(cfl.at[pl.ds(0, _D2)], gath.at[r0], semf).wait()
        return z

    lax.fori_loop(0, n_f + n_l, drainf, 0)


def _gather(unq_inv, xout, recvals, recids):
    mesh = plsc.VectorSubcoreMesh(core_axis_name="c", subcore_axis_name="s")
    f = pl.kernel(
        _gather_body,
        out_type=jax.ShapeDtypeStruct((_N, _D2), jnp.float32),
        mesh=mesh,
        scratch_types=[
            pltpu.VMEM((_CHUNK, _D2), jnp.float32),
            pltpu.VMEM((_CHUNK, _D2), jnp.float32),
            pltpu.VMEM((_CHUNK,), jnp.int32),
            pltpu.VMEM((_CHUNK,), jnp.int32),
            pltpu.VMEM((2 * _NT * _D2,), jnp.float32),
            pltpu.VMEM((2 * _D2,), jnp.float32),
            pltpu.VMEM((_NT * 16,), jnp.int32),
            pltpu.SemaphoreType.DMA,
            pltpu.SemaphoreType.DMA,
            pltpu.SemaphoreType.DMA,
            pltpu.SemaphoreType.DMA,
            pltpu.SemaphoreType.DMA,
            pltpu.SemaphoreType.DMA,
            pltpu.SemaphoreType.DMA,
        ],
        compiler_params=_sc_params(),
    )
    return f(unq_inv, xout, recvals, recids)


def _asm_body(x_ref, g_ref, par_ref, out_ref):
    scale = par_ref[0:1, :]
    shift = par_ref[1:2, :]
    pos = scale >= 0.0
    x = x_ref[...] * scale + shift
    gmin = g_ref[:, 0:_D_HID]
    gmax = g_ref[:, _D_HID:_D2]
    lo = jnp.where(pos, gmin, gmax) * scale + shift
    hi = jnp.where(pos, gmax, gmin) * scale + shift
    out_ref[:, 0:_D_HID] = x
    out_ref[:, _D_HID:_D2] = lo
    out_ref[:, _D2:3 * _D_HID] = hi


def _asm(x_raw, gath, par):
    nblk = _N // _ROWS_BLK
    return pl.pallas_call(
        _asm_body,
        grid=(nblk,),
        in_specs=[
            pl.BlockSpec((_ROWS_BLK, _D_HID), lambda i: (i, 0)),
            pl.BlockSpec((_ROWS_BLK, _D2), lambda i: (i, 0)),
            pl.BlockSpec((2, _D_HID), lambda i: (0, 0)),
        ],
        out_specs=pl.BlockSpec((_ROWS_BLK, 3 * _D_HID), lambda i: (i, 0)),
        out_shape=jax.ShapeDtypeStruct((_N, 3 * _D_HID), jnp.float32),
    )(x_raw, gath, par)


def kernel(inputs, unq_inv, W, gamma, beta):
    ids = unq_inv.astype(jnp.int32)
    x_raw, par = _mm_stats(inputs, W, gamma, beta)
    xout, recvals, recids = _seg_reduce(x_raw.reshape(-1), ids)
    gath = _gather(ids, xout, recvals, recids)
    return _asm(x_raw, gath, par)


# speedup vs baseline: 2.5081x; 1.0342x over previous
import dataclasses
import functools
import jax
import jax.numpy as jnp
from jax import lax
from jax.experimental import pallas as pl
from jax.experimental.pallas import tpu as pltpu
from jax.experimental.pallas import tpu_sc as plsc

_N = 320000
_D_IN = 128
_D_HID = 64
_NUM_SEG = 10000
_EPS = 1e-3

_ROWS_BLK = 6400
_NT = 32
_TROWS = _N // _NT
_CHUNK = 400
_NCHUNK = _TROWS // _CHUNK
_STAGE = 64
_D2 = 2 * _D_HID

_NEG = -3.4028235e38
_POS = 3.4028235e38


def _sc_params():
    cp = pltpu.CompilerParams()
    if "needs_layout_passes" in pltpu.CompilerParams.__dataclass_fields__:
        cp = dataclasses.replace(cp, needs_layout_passes=False)
    return cp


def _mm_body(x_ref, w_ref, g_ref, b_ref, y_ref, par_ref, acc_ref):
    i = pl.program_id(0)
    y = lax.dot_general(x_ref[...], w_ref[...], (((1,), (1,)), ((), ())),
                        preferred_element_type=jnp.float32)
    y_ref[...] = y

    @pl.when(i == 0)
    def _():
        acc_ref[...] = jnp.zeros_like(acc_ref)

    acc_ref[0:1, :] += jnp.sum(y, axis=0, keepdims=True)
    acc_ref[1:2, :] += jnp.sum(y * y, axis=0, keepdims=True)

    @pl.when(i == pl.num_programs(0) - 1)
    def _():
        mean = acc_ref[0:1, :] / _N
        var = acc_ref[1:2, :] / _N - mean * mean
        scale = g_ref[...] * lax.rsqrt(var + _EPS)
        par_ref[0:1, :] = scale
        par_ref[1:2, :] = b_ref[...] - mean * scale


def _mm_stats(inputs, W, gamma, beta):
    nblk = _N // _ROWS_BLK
    return pl.pallas_call(
        _mm_body,
        grid=(nblk,),
        in_specs=[
            pl.BlockSpec((_ROWS_BLK, _D_IN), lambda i: (i, 0)),
            pl.BlockSpec((_D_HID, _D_IN), lambda i: (0, 0)),
            pl.BlockSpec((1, _D_HID), lambda i: (0, 0)),
            pl.BlockSpec((1, _D_HID), lambda i: (0, 0)),
        ],
        out_specs=[
            pl.BlockSpec((_ROWS_BLK, _D_HID), lambda i: (i, 0)),
            pl.BlockSpec((2, _D_HID), lambda i: (0, 0)),
        ],
        out_shape=[
            jax.ShapeDtypeStruct((_N, _D_HID), jnp.float32),
            jax.ShapeDtypeStruct((2, _D_HID), jnp.float32),
        ],
        scratch_shapes=[pltpu.VMEM((2, _D_HID), jnp.float32)],
    )(inputs, W, gamma.reshape(1, -1), beta.reshape(1, -1))


def _seg_reduce_body(xbn, ids, xout, recvals, recids,
                     buf, idsb, stage, rec_v, accv, recid_v, st_sm,
                     sem0, sem1, sem_out, sem_fin):
    t = lax.axis_index("s") * 2 + lax.axis_index("c")
    r0 = t * _TROWS
    _GPC = _CHUNK // 16

    def issue(ci):
        half = lax.rem(ci, 2)
        src_x = xbn.at[pl.ds((r0 + ci * _CHUNK) * _D_HID, _CHUNK * _D_HID)]
        dst_x = buf.at[pl.ds(half * _CHUNK * _D_HID, _CHUNK * _D_HID)]
        src_i = ids.at[pl.ds(r0 + ci * _CHUNK, _CHUNK)]
        dst_i = idsb.at[pl.ds(half * _CHUNK, _CHUNK)]

        @pl.when(half == 0)
        def _():
            pltpu.async_copy(src_x, dst_x, sem0)
            pltpu.async_copy(src_i, dst_i, sem0)

        @pl.when(half == 1)
        def _():
            pltpu.async_copy(src_x, dst_x, sem1)
            pltpu.async_copy(src_i, dst_i, sem1)

    def wait(ci):
        dummy_x = buf.at[pl.ds(0, _CHUNK * _D_HID)]
        dummy_i = idsb.at[pl.ds(0, _CHUNK)]
        src_x = xbn.at[pl.ds(0, _CHUNK * _D_HID)]
        src_i = ids.at[pl.ds(0, _CHUNK)]

        @pl.when(lax.rem(ci, 2) == 0)
        def _():
            pltpu.make_async_copy(src_x, dummy_x, sem0).wait()
            pltpu.make_async_copy(src_i, dummy_i, sem0).wait()

        @pl.when(lax.rem(ci, 2) == 1)
        def _():
            pltpu.make_async_copy(src_x, dummy_x, sem1).wait()
            pltpu.make_async_copy(src_i, dummy_i, sem1).wait()

    issue(jnp.int32(0))
    issue(jnp.int32(1))

    st_sm[0] = jnp.int32(-1)
    st_sm[1] = jnp.int32(0)
    st_sm[2] = jnp.int32(-1)
    for k in range(4):
        accv[pl.ds(16 * k, 16)] = jnp.full((16,), _POS, jnp.float32)
        accv[pl.ds(_D_HID + 16 * k, 16)] = jnp.full((16,), _NEG, jnp.float32)

    def flush_seg(cur, nf, amin, amax):
        m = nf - 1

        @pl.when(nf == 0)
        def _():
            for k in range(4):
                rec_v[pl.ds(16 * k, 16)] = amin[k]
                rec_v[pl.ds(_D_HID + 16 * k, 16)] = amax[k]

        @pl.when(nf > 0)
        def _():
            @pl.when(jnp.logical_and(m > 0, lax.rem(m, _STAGE) == 0))
            def _():
                def dr(i, z):
                    pltpu.make_async_copy(
                        xout.at[0], stage.at[pl.ds(0, _D2)], sem_out).wait()
                    return z
                lax.fori_loop(0, _STAGE, dr, 0)
            jb = lax.rem(jnp.maximum(m, 0), _STAGE) * _D2
            for k in range(4):
                stage[pl.ds(jb + 16 * k, 16)] = amin[k]
                stage[pl.ds(jb + _D_HID + 16 * k, 16)] = amax[k]
            pltpu.async_copy(stage.at[pl.ds(jb, _D2)], xout.at[cur], sem_out)

    def chunk(ci, z):
        wait(ci)
        half = lax.rem(ci, 2)
        ibase = half * _CHUNK
        xbase = half * _CHUNK * _D_HID

        def group(g, zz):
            gib = ibase + g * 16
            gxb = xbase + g * 16 * _D_HID
            idvec = idsb[pl.ds(gib, 16)]
            i0 = idvec[0]
            i15 = idvec[15]
            cur0 = st_sm[0]
            fast = jnp.logical_and(i0 == cur0, i15 == i0)

            @pl.when(fast)
            def _():
                amin = [accv[pl.ds(16 * k, 16)] for k in range(4)]
                amax = [accv[pl.ds(_D_HID + 16 * k, 16)] for k in range(4)]
                for l in range(16):
                    base = gxb + l * _D_HID
                    for k in range(4):
                        xk = buf[pl.ds(base + 16 * k, 16)]
                        amin[k] = jnp.minimum(amin[k], xk)
                        amax[k] = jnp.maximum(amax[k], xk)
                for k in range(4):
                    accv[pl.ds(16 * k, 16)] = amin[k]
                    accv[pl.ds(_D_HID + 16 * k, 16)] = amax[k]

            @pl.when(jnp.logical_not(fast))
            def _():
                cur, nf, fid = st_sm[0], st_sm[1], st_sm[2]
                amin = [accv[pl.ds(16 * k, 16)] for k in range(4)]
                amax = [accv[pl.ds(_D_HID + 16 * k, 16)] for k in range(4)]
                first_g = jnp.logical_and(ci == 0, g == 0)
                for l in range(16):
                    idv = idvec[l]
                    base = gxb + l * _D_HID
                    xs = [buf[pl.ds(base + 16 * k, 16)] for k in range(4)]
                    brk = idv != cur
                    if l == 0:
                        brk = jnp.logical_and(brk, jnp.logical_not(first_g))

                    @pl.when(brk)
                    def _(cur=cur, nf=nf, amin=amin, amax=amax):
                        flush_seg(cur, nf, amin, amax)

                    amin = [jnp.where(brk, xs[k], jnp.minimum(amin[k], xs[k]))
                            for k in range(4)]
                    amax = [jnp.where(brk, xs[k], jnp.maximum(amax[k], xs[k]))
                            for k in range(4)]
                    fid = jnp.where(jnp.logical_and(brk, nf == 0), cur, fid)
                    nf = nf + brk.astype(jnp.int32)
                    cur = idv
                for k in range(4):
                    accv[pl.ds(16 * k, 16)] = amin[k]
                    accv[pl.ds(_D_HID + 16 * k, 16)] = amax[k]
                st_sm[0] = cur
                st_sm[1] = nf
                st_sm[2] = fid
            return zz

        z = lax.fori_loop(0, _GPC, group, z)

        @pl.when(ci + 2 < _NCHUNK)
        def _():
            issue(ci + 2)
        return z

    lax.fori_loop(0, _NCHUNK, chunk, 0)

    cur, nf, fid = st_sm[0], st_sm[1], st_sm[2]
    issued = jnp.maximum(nf - 1, 0)

    for k in range(4):
        rec_v[pl.ds(_D2 + 16 * k, 16)] = accv[pl.ds(16 * k, 16)]
        rec_v[pl.ds(_D2 + _D_HID + 16 * k, 16)] = \
            accv[pl.ds(_D_HID + 16 * k, 16)]
    iota = lax.iota(jnp.int32, 16)
    recid_v[...] = jnp.where(iota == 0, fid, jnp.where(iota == 1, cur, 0))

    ndrain = issued - lax.div(jnp.maximum(issued - 1, 0),
                              jnp.int32(_STAGE)) * _STAGE
    ndrain = jnp.where(issued == 0, 0, ndrain)

    def drain(i, z):
        pltpu.make_async_copy(xout.at[0], stage.at[pl.ds(0, _D2)],
                              sem_out).wait()
        return z

    lax.fori_loop(0, ndrain, drain, 0)

    pltpu.async_copy(rec_v, recvals.at[pl.ds(2 * t * _D2, 2 * _D2)], sem_fin)
    pltpu.async_copy(recid_v, recids.at[pl.ds(t * 16, 16)], sem_fin)
    pltpu.make_async_copy(rec_v, recvals.at[pl.ds(2 * t * _D2, 2 * _D2)],
                          sem_fin).wait()
    pltpu.make_async_copy(recid_v, recids.at[pl.ds(t * 16, 16)],
                          sem_fin).wait()


def _seg_reduce(x_bn_flat, unq_inv):
    mesh = plsc.VectorSubcoreMesh(core_axis_name="c", subcore_axis_name="s")
    f = pl.kernel(
        _seg_reduce_body,
        out_type=[
            jax.ShapeDtypeStruct((_NUM_SEG, _D2), jnp.float32),
            jax.ShapeDtypeStruct((2 * _NT * _D2,), jnp.float32),
            jax.ShapeDtypeStruct((_NT * 16,), jnp.int32),
        ],
        mesh=mesh,
        scratch_types=[
            pltpu.VMEM((2 * _CHUNK * _D_HID,), jnp.float32),
            pltpu.VMEM((2 * _CHUNK,), jnp.int32),
            pltpu.VMEM((_STAGE * _D2,), jnp.float32),
            pltpu.VMEM((2 * _D2,), jnp.float32),
            pltpu.VMEM((_D2,), jnp.float32),
            pltpu.VMEM((16,), jnp.int32),
            pltpu.SMEM((8,), jnp.int32),
            pltpu.SemaphoreType.DMA,
            pltpu.SemaphoreType.DMA,
            pltpu.SemaphoreType.DMA,
            pltpu.SemaphoreType.DMA,
        ],
        compiler_params=_sc_params(),
    )
    return f(x_bn_flat, unq_inv)


def _gather_body(ids, xout, recvals, recids, gath,
                 gbuf0, gbuf1, idsv0, idsv1, rv, cfl, recidv,
                 semg0, semg1, semi0, semi1, semo, semr, semf):
    t = lax.axis_index("s") * 2 + lax.axis_index("c")
    r0 = t * _TROWS
    gbufs = (gbuf0, gbuf1)
    idsvs = (idsv0, idsv1)
    semgs = (semg0, semg1)
    semis = (semi0, semi1)

    pltpu.async_copy(recvals, rv, semr)
    pltpu.async_copy(recids, recidv, semr)
    pltpu.make_async_copy(recvals, rv, semr).wait()
    pltpu.make_async_copy(recids, recidv, semr).wait()

    myrec = recidv[pl.ds(t * 16, 16)]
    my_first = myrec[0]
    my_first = jnp.where(my_first == -1, myrec[1], my_first)
    my_last = myrec[1]

    for k in range(4):
        cfl[pl.ds(16 * k, 16)] = jnp.full((16,), _POS, jnp.float32)
        cfl[pl.ds(_D_HID + 16 * k, 16)] = jnp.full((16,), _NEG, jnp.float32)
        cfl[pl.ds(_D2 + 16 * k, 16)] = jnp.full((16,), _POS, jnp.float32)
        cfl[pl.ds(_D2 + _D_HID + 16 * k, 16)] = jnp.full((16,), _NEG,
                                                         jnp.float32)

    def comb(u, z):
        uvec = recidv[pl.ds(u * 16, 16)]
        for pos in range(2):
            uid = uvec[pos]
            ub = (2 * u + pos) * _D2
            for rowb, myid in ((0, my_first), (_D2, my_last)):
                @pl.when(uid == myid)
                def _(ub=ub, rowb=rowb):
                    for k in range(4):
                        cfl[pl.ds(rowb + 16 * k, 16)] = jnp.minimum(
                            cfl[pl.ds(rowb + 16 * k, 16)],
                            rv[pl.ds(ub + 16 * k, 16)])
                        cfl[pl.ds(rowb + _D_HID + 16 * k, 16)] = jnp.maximum(
                            cfl[pl.ds(rowb + _D_HID + 16 * k, 16)],
                            rv[pl.ds(ub + _D_HID + 16 * k, 16)])
        return z

    lax.fori_loop(0, _NT, comb, 0)

    def issue(ci):
        b = ci % 2
        base = r0 + ci * _CHUNK
        pltpu.async_copy(ids.at[pl.ds(base, _CHUNK)], idsvs[b], semis[b])

    def wait_ids(ci):
        b = ci % 2
        pltpu.make_async_copy(ids.at[pl.ds(r0, _CHUNK)], idsvs[b],
                              semis[b]).wait()

    def gather(ci):
        b = ci % 2
        for off, ln in ((0, 128), (128, 128), (256, 128), (384, 16)):
            pltpu.async_copy(xout.at[idsvs[b].at[pl.ds(off, ln)]],
                             gbufs[b].at[pl.ds(off, ln)], semgs[b])

    def wait_gather(ci):
        b = ci % 2
        for off, ln in ((0, 128), (128, 128), (256, 128), (384, 16)):
            pltpu.make_async_copy(
                xout.at[idsvs[b].at[pl.ds(off, ln)]],
                gbufs[b].at[pl.ds(off, ln)], semgs[b]).wait()

    issue(0)
    wait_ids(0)
    gather(0)
    issue(1)

    accf = jnp.zeros((16,), jnp.int32)
    accl = jnp.zeros((16,), jnp.int32)

    for ci in range(_NCHUNK):
        b = ci % 2
        base = r0 + ci * _CHUNK
        gbuf = gbufs[b]
        idsv = idsvs[b]

        def cnt(g, acc, idsv=idsv):
            af, al = acc
            idg = idsv[pl.ds(16 * g, 16)]
            af = af + (idg == my_first).astype(jnp.int32)
            al = al + (idg == my_last).astype(jnp.int32)
            return (af, al)

        accf, accl = lax.fori_loop(0, _CHUNK // 16, cnt, (accf, accl))

        wait_gather(ci)
        if ci + 1 < _NCHUNK:
            wait_ids(ci + 1)
            gather(ci + 1)
        if ci + 2 < _NCHUNK:
            issue(ci + 2)

        pltpu.async_copy(gbuf, gath.at[pl.ds(base, _CHUNK)], semo)
        if ci >= 1:
            pltpu.make_async_copy(
                gbufs[(ci + 1) % 2], gath.at[pl.ds(r0, _CHUNK)], semo).wait()

    pltpu.make_async_copy(
        gbufs[(_NCHUNK - 1) % 2], gath.at[pl.ds(r0, _CHUNK)], semo).wait()

    n_f = jnp.sum(accf)
    n_l = jnp.sum(accl)

    def fixf(i, z):
        pltpu.async_copy(cfl.at[pl.ds(0, _D2)], gath.at[r0 + i], semf)
        return z

    lax.fori_loop(0, n_f, fixf, 0)

    def fixl(i, z):
        pltpu.async_copy(cfl.at[pl.ds(_D2, _D2)],
                         gath.at[r0 + _TROWS - 1 - i], semf)
        return z

    lax.fori_loop(0, n_l, fixl, 0)

    def drainf(i, z):
        pltpu.make_async_copy(cfl.at[pl.ds(0, _D2)], gath.at[r0], semf).wait()
        return z

    lax.fori_loop(0, n_f + n_l, drainf, 0)


def _gather(unq_inv, xout, recvals, recids):
    mesh = plsc.VectorSubcoreMesh(core_axis_name="c", subcore_axis_name="s")
    f = pl.kernel(
        _gather_body,
        out_type=jax.ShapeDtypeStruct((_N, _D2), jnp.float32),
        mesh=mesh,
        scratch_types=[
            pltpu.VMEM((_CHUNK, _D2), jnp.float32),
            pltpu.VMEM((_CHUNK, _D2), jnp.float32),
            pltpu.VMEM((_CHUNK,), jnp.int32),
            pltpu.VMEM((_CHUNK,), jnp.int32),
            pltpu.VMEM((2 * _NT * _D2,), jnp.float32),
            pltpu.VMEM((2 * _D2,), jnp.float32),
            pltpu.VMEM((_NT * 16,), jnp.int32),
            pltpu.SemaphoreType.DMA,
            pltpu.SemaphoreType.DMA,
            pltpu.SemaphoreType.DMA,
            pltpu.SemaphoreType.DMA,
            pltpu.SemaphoreType.DMA,
            pltpu.SemaphoreType.DMA,
            pltpu.SemaphoreType.DMA,
        ],
        compiler_params=_sc_params(),
    )
    return f(unq_inv, xout, recvals, recids)


def _asm_body(x_ref, g_ref, par_ref, out_ref):
    scale = par_ref[0:1, :]
    shift = par_ref[1:2, :]
    pos = scale >= 0.0
    x = x_ref[...] * scale + shift
    gmin = g_ref[:, 0:_D_HID]
    gmax = g_ref[:, _D_HID:_D2]
    lo = jnp.where(pos, gmin, gmax) * scale + shift
    hi = jnp.where(pos, gmax, gmin) * scale + shift
    out_ref[:, 0:_D_HID] = x
    out_ref[:, _D_HID:_D2] = lo
    out_ref[:, _D2:3 * _D_HID] = hi


def _asm(x_raw, gath, par):
    nblk = _N // _ROWS_BLK
    return pl.pallas_call(
        _asm_body,
        grid=(nblk,),
        in_specs=[
            pl.BlockSpec((_ROWS_BLK, _D_HID), lambda i: (i, 0)),
            pl.BlockSpec((_ROWS_BLK, _D2), lambda i: (i, 0)),
            pl.BlockSpec((2, _D_HID), lambda i: (0, 0)),
        ],
        out_specs=pl.BlockSpec((_ROWS_BLK, 3 * _D_HID), lambda i: (i, 0)),
        out_shape=jax.ShapeDtypeStruct((_N, 3 * _D_HID), jnp.float32),
    )(x_raw, gath, par)


def kernel(inputs, unq_inv, W, gamma, beta):
    ids = unq_inv.astype(jnp.int32)
    x_raw, par = _mm_stats(inputs, W, gamma, beta)
    xout, recvals, recids = _seg_reduce(x_raw.reshape(-1), ids)
    gath = _gather(ids, xout, recvals, recids)
    return _asm(x_raw, gath, par)
